# Initial kernel scaffold; baseline (speedup 1.0000x reference)
import jax
import jax.numpy as jnp
from jax.experimental import pallas as pl


def kernel(x, params, edge_index, batch):
    raise NotImplementedError("write your pallas kernel here")


# speedup vs baseline: 4.4190x; 4.4190x over previous
import functools

import jax
import jax.numpy as jnp
from jax import lax
from jax.experimental import pallas as pl
from jax.experimental.pallas import tpu as pltpu
from jax.experimental.pallas import tpu_sc as plsc

NC = 2
NS = 16
CHUNK = 80


@functools.lru_cache(maxsize=None)
def _make_sc_agg(n, d, e):
    assert e % (NC * NS * CHUNK) == 0
    ept = e // (NC * NS)
    nsteps = ept // CHUNK
    rs = (n // NS) // 8 * 8
    tail = n - NS * rs

    def body(h_hbm, src_hbm, dst_hbm, out_hbm, agg_s, sidx_v, didx_v,
             rows_v, sem):
        c = lax.axis_index("c")
        s = lax.axis_index("s")
        pltpu.sync_copy(h_hbm.at[pl.ds(s * rs, rs)], agg_s.at[pl.ds(s * rs, rs)])
        if tail:
            @pl.when(s == 0)
            def _():
                pltpu.sync_copy(h_hbm.at[pl.ds(NS * rs, tail)],
                                agg_s.at[pl.ds(NS * rs, tail)])
        plsc.subcore_barrier()
        ebase = c * (e // NC) + s * ept

        def step(j, carry):
            off = ebase + j * CHUNK
            pltpu.sync_copy(src_hbm.at[pl.ds(off, CHUNK)], sidx_v)
            pltpu.sync_copy(dst_hbm.at[pl.ds(off, CHUNK)], didx_v)
            pltpu.async_copy(h_hbm.at[sidx_v], rows_v, sem).wait()
            pltpu.sync_copy(rows_v, agg_s.at[didx_v], add=True)
            return carry

        lax.fori_loop(0, nsteps, step, 0)
        plsc.subcore_barrier()
        pltpu.sync_copy(agg_s.at[pl.ds(s * rs, rs)],
                        out_hbm.at[c].at[pl.ds(s * rs, rs)])
        if tail:
            @pl.when(s == 0)
            def _():
                pltpu.sync_copy(agg_s.at[pl.ds(NS * rs, tail)],
                                out_hbm.at[c].at[pl.ds(NS * rs, tail)])

    mesh = plsc.VectorSubcoreMesh(core_axis_name="c", subcore_axis_name="s",
                                  num_cores=NC, num_subcores=NS)
    return pl.kernel(
        body,
        out_type=jax.ShapeDtypeStruct((NC, n, d), jnp.float32),
        mesh=mesh,
        scratch_types=[
            pltpu.VMEM_SHARED((n, d), jnp.float32),
            pltpu.VMEM((CHUNK,), jnp.int32),
            pltpu.VMEM((CHUNK,), jnp.int32),
            pltpu.VMEM((CHUNK, d), jnp.float32),
            pltpu.SemaphoreType.DMA,
        ],
    )


def _bn_relu(z, g, b):
    mu = jnp.mean(z, axis=0, keepdims=True)
    zc = z - mu
    var = jnp.mean(zc * zc, axis=0, keepdims=True)
    return jnp.maximum(zc / jnp.sqrt(var + 1e-5) * g + b, 0.0)


def _tc_layer_body(h_b, p_b, w1_b, b1_b, g1_b, be1_b, w2_b, b2_b, g2_b,
                   be2_b, bat_b, hout_b, ro_b):
    h = h_b[...]
    z = p_b[0] + p_b[1] - h
    z = jnp.dot(z, w1_b[...], preferred_element_type=jnp.float32, precision=lax.Precision.HIGHEST) + b1_b[...]
    z = _bn_relu(z, g1_b[...], be1_b[...])
    z = jnp.dot(z, w2_b[...], preferred_element_type=jnp.float32, precision=lax.Precision.HIGHEST) + b2_b[...]
    hn = _bn_relu(z, g2_b[...], be2_b[...])
    hout_b[...] = hn
    g = ro_b.shape[0]
    seg = lax.broadcasted_iota(jnp.int32, (g, h.shape[0]), 0)
    onehot = (seg == bat_b[...]).astype(jnp.float32)
    ro_b[...] = jnp.dot(onehot, hn, preferred_element_type=jnp.float32, precision=lax.Precision.HIGHEST)


@functools.lru_cache(maxsize=None)
def _make_tc_layer(n, d, hdim, g):
    return pl.pallas_call(
        _tc_layer_body,
        out_shape=(jax.ShapeDtypeStruct((n, hdim), jnp.float32),
                   jax.ShapeDtypeStruct((g, hdim), jnp.float32)),
    )


def _head_body(r0_b, r1_b, r2_b, wc1_b, bc1_b, wc2_b, bc2_b, out_b):
    hdim = r0_b.shape[1]
    hc = (jnp.dot(r0_b[...], wc1_b[0:hdim], preferred_element_type=jnp.float32, precision=lax.Precision.HIGHEST)
          + jnp.dot(r1_b[...], wc1_b[hdim:2 * hdim],
                    preferred_element_type=jnp.float32, precision=lax.Precision.HIGHEST)
          + jnp.dot(r2_b[...], wc1_b[2 * hdim:3 * hdim],
                    preferred_element_type=jnp.float32, precision=lax.Precision.HIGHEST))
    hc = jnp.maximum(hc + bc1_b[...], 0.0)
    out_b[...] = (jnp.dot(hc, wc2_b[...], preferred_element_type=jnp.float32, precision=lax.Precision.HIGHEST)
                  + bc2_b[...])


@functools.lru_cache(maxsize=None)
def _make_head(g, c):
    return pl.pallas_call(
        _head_body,
        out_shape=jax.ShapeDtypeStruct((g, c), jnp.float32),
    )


def kernel(x, params, edge_index, batch):
    n, d = x.shape
    e = edge_index.shape[1]
    g = 64
    nlayers = sum(1 for k in params if k.startswith('layer'))
    c_out = params['Wc2'].shape[1]
    hdim = params['layer0']['W2'].shape[1]

    sc_agg = _make_sc_agg(n, d, e)
    tc_layer = _make_tc_layer(n, d, hdim, g)
    head = _make_head(g, c_out)

    bat2d = batch.reshape(1, n)
    h = x
    ros = []
    for i in range(nlayers):
        p = params['layer%d' % i]
        part = sc_agg(h, edge_index[0], edge_index[1])
        h, ro = tc_layer(h, part,
                         p['W1'], p['b1'].reshape(1, -1),
                         p['g1'].reshape(1, -1), p['be1'].reshape(1, -1),
                         p['W2'], p['b2'].reshape(1, -1),
                         p['g_out'].reshape(1, -1), p['be_out'].reshape(1, -1),
                         bat2d)
        ros.append(ro)
    return head(ros[0], ros[1], ros[2],
                params['Wc1'], params['bc1'].reshape(1, -1),
                params['Wc2'], params['bc2'].reshape(1, -1))


# speedup vs baseline: 7.9994x; 1.8102x over previous
import functools

import jax
import jax.numpy as jnp
from jax import lax
from jax.experimental import pallas as pl
from jax.experimental.pallas import tpu as pltpu
from jax.experimental.pallas import tpu_sc as plsc

NC = 2
NS = 16
CHUNK = 25
K = 4
NIG = 4


@functools.lru_cache(maxsize=None)
def _make_sc_agg(n, d, e):
    ept = e // (NC * NS)
    nrounds = ept // (K * CHUNK)
    assert ept == nrounds * K * CHUNK and nrounds % NIG == 0 and nrounds >= 8
    rs = (n // NS) // 8 * 8
    tail = n - NS * rs

    def body(h_hbm, idx5_hbm, out_hbm, agg_s, idx_v, rows_v, isem, gsem, ssem):
        c = lax.axis_index("c")
        s = lax.axis_index("s")
        wid = c * NS + s

        def fire_idx(r):
            pltpu.async_copy(idx5_hbm.at[wid].at[r], idx_v.at[r % NIG], isem)

        def wait_idx(r):
            pltpu.make_async_copy(idx5_hbm.at[wid].at[0], idx_v.at[r % NIG],
                                  isem).wait()

        def fire_gathers(r, ig):
            for b in range(K):
                pltpu.async_copy(h_hbm.at[idx_v.at[ig].at[0].at[b]],
                                 rows_v.at[r % 2].at[b], gsem)

        def drain_gathers(p):
            for b in range(K):
                pltpu.make_async_copy(h_hbm.at[idx_v.at[0].at[0].at[0]],
                                      rows_v.at[p].at[b], gsem).wait()

        def fire_scatters(r, ig):
            for b in range(K):
                pltpu.async_copy(rows_v.at[r % 2].at[b],
                                 agg_s.at[idx_v.at[ig].at[1].at[b]], ssem,
                                 add=True)

        def drain_scatters(p):
            for b in range(K):
                pltpu.make_async_copy(rows_v.at[p].at[b],
                                      agg_s.at[idx_v.at[0].at[1].at[0]],
                                      ssem).wait()

        fire_idx(0)
        pltpu.sync_copy(h_hbm.at[pl.ds(s * rs, rs)], agg_s.at[pl.ds(s * rs, rs)])
        if tail:
            @pl.when(s == 0)
            def _():
                pltpu.sync_copy(h_hbm.at[pl.ds(NS * rs, tail)],
                                agg_s.at[pl.ds(NS * rs, tail)])
        wait_idx(0)
        fire_idx(1)
        fire_gathers(0, 0)
        plsc.subcore_barrier()

        def outer(k4, carry):
            for q in range(NIG):
                r = k4 * NIG + q
                drain_gathers(q % 2)
                @pl.when(r >= 1)
                def _():
                    drain_scatters(1 - q % 2)
                @pl.when(r + 1 < nrounds)
                def _():
                    wait_idx(r + 1)
                    @pl.when(r + 2 < nrounds)
                    def _():
                        fire_idx(r + 2)
                    fire_gathers(r + 1, (q + 1) % NIG)
                fire_scatters(r, q)
            return carry

        lax.fori_loop(0, nrounds // NIG, outer, 0)
        drain_scatters(1)
        plsc.subcore_barrier()
        pltpu.sync_copy(agg_s.at[pl.ds(s * rs, rs)],
                        out_hbm.at[c].at[pl.ds(s * rs, rs)])
        if tail:
            @pl.when(s == 0)
            def _():
                pltpu.sync_copy(agg_s.at[pl.ds(NS * rs, tail)],
                                out_hbm.at[c].at[pl.ds(NS * rs, tail)])

    mesh = plsc.VectorSubcoreMesh(core_axis_name="c", subcore_axis_name="s",
                                  num_cores=NC, num_subcores=NS)
    return pl.kernel(
        body,
        out_type=jax.ShapeDtypeStruct((NC, n, d), jnp.float32),
        mesh=mesh,
        scratch_types=[
            pltpu.VMEM_SHARED((n, d), jnp.float32),
            pltpu.VMEM((NIG, 2, K, CHUNK), jnp.int32),
            pltpu.VMEM((2, K, CHUNK, d), jnp.float32),
            pltpu.SemaphoreType.DMA,
            pltpu.SemaphoreType.DMA,
            pltpu.SemaphoreType.DMA,
        ],
    )


def _bn_relu(z, g, b):
    mu = jnp.mean(z, axis=0, keepdims=True)
    zc = z - mu
    var = jnp.mean(zc * zc, axis=0, keepdims=True)
    return jnp.maximum(zc / jnp.sqrt(var + 1e-5) * g + b, 0.0)


def _tc_layer_body(h_b, p_b, w1_b, b1_b, g1_b, be1_b, w2_b, b2_b, g2_b,
                   be2_b, bat_b, hout_b, ro_b):
    h = h_b[...]
    z = p_b[0] + p_b[1] - h
    z = jnp.dot(z, w1_b[...], preferred_element_type=jnp.float32, precision=lax.Precision.HIGHEST) + b1_b[...]
    z = _bn_relu(z, g1_b[...], be1_b[...])
    z = jnp.dot(z, w2_b[...], preferred_element_type=jnp.float32, precision=lax.Precision.HIGHEST) + b2_b[...]
    hn = _bn_relu(z, g2_b[...], be2_b[...])
    hout_b[...] = hn
    g = ro_b.shape[0]
    seg = lax.broadcasted_iota(jnp.int32, (g, h.shape[0]), 0)
    onehot = (seg == bat_b[...]).astype(jnp.float32)
    ro_b[...] = jnp.dot(onehot, hn, preferred_element_type=jnp.float32, precision=lax.Precision.HIGHEST)


@functools.lru_cache(maxsize=None)
def _make_tc_layer(n, d, hdim, g):
    return pl.pallas_call(
        _tc_layer_body,
        out_shape=(jax.ShapeDtypeStruct((n, hdim), jnp.float32),
                   jax.ShapeDtypeStruct((g, hdim), jnp.float32)),
    )


def _head_body(r0_b, r1_b, r2_b, wc1_b, bc1_b, wc2_b, bc2_b, out_b):
    hdim = r0_b.shape[1]
    hc = (jnp.dot(r0_b[...], wc1_b[0:hdim], preferred_element_type=jnp.float32, precision=lax.Precision.HIGHEST)
          + jnp.dot(r1_b[...], wc1_b[hdim:2 * hdim],
                    preferred_element_type=jnp.float32, precision=lax.Precision.HIGHEST)
          + jnp.dot(r2_b[...], wc1_b[2 * hdim:3 * hdim],
                    preferred_element_type=jnp.float32, precision=lax.Precision.HIGHEST))
    hc = jnp.maximum(hc + bc1_b[...], 0.0)
    out_b[...] = (jnp.dot(hc, wc2_b[...], preferred_element_type=jnp.float32, precision=lax.Precision.HIGHEST)
                  + bc2_b[...])


@functools.lru_cache(maxsize=None)
def _make_head(g, c):
    return pl.pallas_call(
        _head_body,
        out_shape=jax.ShapeDtypeStruct((g, c), jnp.float32),
    )


def kernel(x, params, edge_index, batch):
    n, d = x.shape
    e = edge_index.shape[1]
    g = 64
    nlayers = sum(1 for k in params if k.startswith('layer'))
    c_out = params['Wc2'].shape[1]
    hdim = params['layer0']['W2'].shape[1]

    sc_agg = _make_sc_agg(n, d, e)
    tc_layer = _make_tc_layer(n, d, hdim, g)
    head = _make_head(g, c_out)

    bat2d = batch.reshape(1, n)
    nrounds = e // (NC * NS) // (K * CHUNK)
    idx5 = jnp.transpose(
        edge_index.reshape(2, NC * NS, nrounds, K, CHUNK), (1, 2, 0, 3, 4))
    h = x
    ros = []
    for i in range(nlayers):
        p = params['layer%d' % i]
        part = sc_agg(h, idx5)
        h, ro = tc_layer(h, part,
                         p['W1'], p['b1'].reshape(1, -1),
                         p['g1'].reshape(1, -1), p['be1'].reshape(1, -1),
                         p['W2'], p['b2'].reshape(1, -1),
                         p['g_out'].reshape(1, -1), p['be_out'].reshape(1, -1),
                         bat2d)
        ros.append(ro)
    return head(ros[0], ros[1], ros[2],
                params['Wc1'], params['bc1'].reshape(1, -1),
                params['Wc2'], params['bc2'].reshape(1, -1))


# speedup vs baseline: 8.9935x; 1.1243x over previous
import functools

import jax
import jax.numpy as jnp
from jax import lax
from jax.experimental import pallas as pl
from jax.experimental.pallas import tpu as pltpu
from jax.experimental.pallas import tpu_sc as plsc

NC = 2
NS = 16
CHUNK = 50
NRB = 4
NIG = 8


@functools.lru_cache(maxsize=None)
def _make_sc_agg(n, d, e):
    ept = e // (NC * NS)
    nrounds = ept // CHUNK
    assert ept == nrounds * CHUNK and nrounds % NIG == 0 and nrounds >= 2 * NIG
    rs = (n // NS) // 8 * 8
    tail = n - NS * rs

    def body(h_hbm, idx4_hbm, out_hbm, agg_s, idx_v, rows_v, isem, gsem, ssem):
        c = lax.axis_index("c")
        s = lax.axis_index("s")
        wid = c * NS + s

        def fire_idx(r, u):
            pltpu.async_copy(idx4_hbm.at[wid].at[r], idx_v.at[u % NIG],
                             isem.at[u % NIG])

        def wait_idx(u):
            pltpu.make_async_copy(idx4_hbm.at[wid].at[0], idx_v.at[u % NIG],
                                  isem.at[u % NIG]).wait()

        def fire_gather(r, u):
            pltpu.async_copy(h_hbm.at[idx_v.at[u % NIG].at[0]],
                             rows_v.at[u % NRB], gsem.at[u % NRB])

        def wait_gather(u):
            pltpu.make_async_copy(h_hbm.at[idx_v.at[0].at[0]],
                                  rows_v.at[u % NRB], gsem.at[u % NRB]).wait()

        def fire_scatter(u):
            pltpu.async_copy(rows_v.at[u % NRB],
                             agg_s.at[idx_v.at[u % NIG].at[1]],
                             ssem.at[u % NRB], add=True)

        def wait_scatter(u):
            pltpu.make_async_copy(rows_v.at[u % NRB],
                                  agg_s.at[idx_v.at[0].at[1]],
                                  ssem.at[u % NRB]).wait()

        for q in range(4):
            fire_idx(q, q)
        pltpu.sync_copy(h_hbm.at[pl.ds(s * rs, rs)], agg_s.at[pl.ds(s * rs, rs)])
        if tail:
            @pl.when(s == 0)
            def _():
                pltpu.sync_copy(h_hbm.at[pl.ds(NS * rs, tail)],
                                agg_s.at[pl.ds(NS * rs, tail)])
        for q in range(2):
            wait_idx(q)
            fire_gather(q, q)
        plsc.subcore_barrier()

        def outer(k8, carry):
            for u in range(NIG):
                r = k8 * NIG + u
                wait_gather(u)
                fire_scatter(u)
                @pl.when(r + 4 < nrounds)
                def _():
                    fire_idx(r + 4, u + 4)
                @pl.when(r + 2 < nrounds)
                def _():
                    wait_idx(u + 2)
                    @pl.when(r >= 2)
                    def _():
                        wait_scatter(u + 2)
                    fire_gather(r + 2, u + 2)
            return carry

        lax.fori_loop(0, nrounds // NIG, outer, 0)
        for q in range(NRB):
            wait_scatter(q)
        plsc.subcore_barrier()
        pltpu.sync_copy(agg_s.at[pl.ds(s * rs, rs)],
                        out_hbm.at[c].at[pl.ds(s * rs, rs)])
        if tail:
            @pl.when(s == 0)
            def _():
                pltpu.sync_copy(agg_s.at[pl.ds(NS * rs, tail)],
                                out_hbm.at[c].at[pl.ds(NS * rs, tail)])

    mesh = plsc.VectorSubcoreMesh(core_axis_name="c", subcore_axis_name="s",
                                  num_cores=NC, num_subcores=NS)
    return pl.kernel(
        body,
        out_type=jax.ShapeDtypeStruct((NC, n, d), jnp.float32),
        mesh=mesh,
        scratch_types=[
            pltpu.VMEM_SHARED((n, d), jnp.float32),
            pltpu.VMEM((NIG, 2, CHUNK), jnp.int32),
            pltpu.VMEM((NRB, CHUNK, d), jnp.float32),
            pltpu.SemaphoreType.DMA((NIG,)),
            pltpu.SemaphoreType.DMA((NRB,)),
            pltpu.SemaphoreType.DMA((NRB,)),
        ],
    )


def _bn_relu(z, g, b):
    mu = jnp.mean(z, axis=0, keepdims=True)
    zc = z - mu
    var = jnp.mean(zc * zc, axis=0, keepdims=True)
    return jnp.maximum(zc / jnp.sqrt(var + 1e-5) * g + b, 0.0)


def _tc_layer_body(h_b, p_b, w1_b, b1_b, g1_b, be1_b, w2_b, b2_b, g2_b,
                   be2_b, bat_b, hout_b, ro_b):
    h = h_b[...]
    z = p_b[0] + p_b[1] - h
    z = jnp.dot(z, w1_b[...], preferred_element_type=jnp.float32, precision=lax.Precision.HIGHEST) + b1_b[...]
    z = _bn_relu(z, g1_b[...], be1_b[...])
    z = jnp.dot(z, w2_b[...], preferred_element_type=jnp.float32, precision=lax.Precision.HIGHEST) + b2_b[...]
    hn = _bn_relu(z, g2_b[...], be2_b[...])
    hout_b[...] = hn
    g = ro_b.shape[0]
    seg = lax.broadcasted_iota(jnp.int32, (g, h.shape[0]), 0)
    onehot = (seg == bat_b[...]).astype(jnp.float32)
    ro_b[...] = jnp.dot(onehot, hn, preferred_element_type=jnp.float32, precision=lax.Precision.HIGHEST)


@functools.lru_cache(maxsize=None)
def _make_tc_layer(n, d, hdim, g):
    return pl.pallas_call(
        _tc_layer_body,
        out_shape=(jax.ShapeDtypeStruct((n, hdim), jnp.float32),
                   jax.ShapeDtypeStruct((g, hdim), jnp.float32)),
    )


def _head_body(r0_b, r1_b, r2_b, wc1_b, bc1_b, wc2_b, bc2_b, out_b):
    hdim = r0_b.shape[1]
    hc = (jnp.dot(r0_b[...], wc1_b[0:hdim], preferred_element_type=jnp.float32, precision=lax.Precision.HIGHEST)
          + jnp.dot(r1_b[...], wc1_b[hdim:2 * hdim],
                    preferred_element_type=jnp.float32, precision=lax.Precision.HIGHEST)
          + jnp.dot(r2_b[...], wc1_b[2 * hdim:3 * hdim],
                    preferred_element_type=jnp.float32, precision=lax.Precision.HIGHEST))
    hc = jnp.maximum(hc + bc1_b[...], 0.0)
    out_b[...] = (jnp.dot(hc, wc2_b[...], preferred_element_type=jnp.float32, precision=lax.Precision.HIGHEST)
                  + bc2_b[...])


@functools.lru_cache(maxsize=None)
def _make_head(g, c):
    return pl.pallas_call(
        _head_body,
        out_shape=jax.ShapeDtypeStruct((g, c), jnp.float32),
    )


def kernel(x, params, edge_index, batch):
    n, d = x.shape
    e = edge_index.shape[1]
    g = 64
    nlayers = sum(1 for k in params if k.startswith('layer'))
    c_out = params['Wc2'].shape[1]
    hdim = params['layer0']['W2'].shape[1]

    sc_agg = _make_sc_agg(n, d, e)
    tc_layer = _make_tc_layer(n, d, hdim, g)
    head = _make_head(g, c_out)

    bat2d = batch.reshape(1, n)
    nrounds = e // (NC * NS) // CHUNK
    idx4 = jnp.transpose(
        edge_index.reshape(2, NC * NS, nrounds, CHUNK), (1, 2, 0, 3))
    h = x
    ros = []
    for i in range(nlayers):
        p = params['layer%d' % i]
        part = sc_agg(h, idx4)
        h, ro = tc_layer(h, part,
                         p['W1'], p['b1'].reshape(1, -1),
                         p['g1'].reshape(1, -1), p['be1'].reshape(1, -1),
                         p['W2'], p['b2'].reshape(1, -1),
                         p['g_out'].reshape(1, -1), p['be_out'].reshape(1, -1),
                         bat2d)
        ros.append(ro)
    return head(ros[0], ros[1], ros[2],
                params['Wc1'], params['bc1'].reshape(1, -1),
                params['Wc2'], params['bc2'].reshape(1, -1))


# speedup vs baseline: 10.0475x; 1.1172x over previous
import functools

import jax
import jax.numpy as jnp
from jax import lax
from jax.experimental import pallas as pl
from jax.experimental.pallas import tpu as pltpu
from jax.experimental.pallas import tpu_sc as plsc

NC = 2
NS = 16
CHUNK = 50
NRB = 4
NIG = 8


@functools.lru_cache(maxsize=None)
def _make_sc_agg(n, d, e):
    ept = e // (NC * NS)
    nrounds = ept // CHUNK
    assert ept == nrounds * CHUNK and nrounds % NIG == 0 and nrounds >= 2 * NIG
    rs = (n // NS) // 8 * 8
    tail = n - NS * rs

    def body(h_hbm, idx4_hbm, out_hbm, agg_s, idx_v, rows_v, isem, gsem, ssem):
        c = lax.axis_index("c")
        s = lax.axis_index("s")
        wid = c * NS + s

        def fire_idx(r, u):
            pltpu.async_copy(idx4_hbm.at[wid].at[r], idx_v.at[u % NIG],
                             isem.at[u % NIG])

        def wait_idx(u):
            pltpu.make_async_copy(idx4_hbm.at[wid].at[0], idx_v.at[u % NIG],
                                  isem.at[u % NIG]).wait()

        def fire_gather(r, u):
            pltpu.async_copy(h_hbm.at[idx_v.at[u % NIG].at[0]],
                             rows_v.at[u % NRB], gsem.at[u % NRB])

        def wait_gather(u):
            pltpu.make_async_copy(h_hbm.at[idx_v.at[0].at[0]],
                                  rows_v.at[u % NRB], gsem.at[u % NRB]).wait()

        def fire_scatter(u):
            pltpu.async_copy(rows_v.at[u % NRB],
                             agg_s.at[idx_v.at[u % NIG].at[1]],
                             ssem.at[u % NRB], add=True)

        def wait_scatter(u):
            pltpu.make_async_copy(rows_v.at[u % NRB],
                                  agg_s.at[idx_v.at[0].at[1]],
                                  ssem.at[u % NRB]).wait()

        for q in range(4):
            fire_idx(q, q)
        pltpu.sync_copy(h_hbm.at[pl.ds(s * rs, rs)], agg_s.at[pl.ds(s * rs, rs)])
        if tail:
            @pl.when(s == 0)
            def _():
                pltpu.sync_copy(h_hbm.at[pl.ds(NS * rs, tail)],
                                agg_s.at[pl.ds(NS * rs, tail)])
        for q in range(2):
            wait_idx(q)
            fire_gather(q, q)
        plsc.subcore_barrier()

        def outer(k8, carry):
            for u in range(NIG):
                r = k8 * NIG + u
                wait_gather(u)
                fire_scatter(u)
                @pl.when(r + 4 < nrounds)
                def _():
                    fire_idx(r + 4, u + 4)
                @pl.when(r + 2 < nrounds)
                def _():
                    wait_idx(u + 2)
                    @pl.when(r >= 2)
                    def _():
                        wait_scatter(u + 2)
                    fire_gather(r + 2, u + 2)
            return carry

        lax.fori_loop(0, nrounds // NIG, outer, 0)
        for q in range(NRB):
            wait_scatter(q)
        plsc.subcore_barrier()
        pltpu.sync_copy(agg_s.at[pl.ds(s * rs, rs)],
                        out_hbm.at[c].at[pl.ds(s * rs, rs)])
        if tail:
            @pl.when(s == 0)
            def _():
                pltpu.sync_copy(agg_s.at[pl.ds(NS * rs, tail)],
                                out_hbm.at[c].at[pl.ds(NS * rs, tail)])

    mesh = plsc.VectorSubcoreMesh(core_axis_name="c", subcore_axis_name="s",
                                  num_cores=NC, num_subcores=NS)
    return pl.kernel(
        body,
        out_type=jax.ShapeDtypeStruct((NC, n, d), jnp.float32),
        mesh=mesh,
        scratch_types=[
            pltpu.VMEM_SHARED((n, d), jnp.float32),
            pltpu.VMEM((NIG, 2, CHUNK), jnp.int32),
            pltpu.VMEM((NRB, CHUNK, d), jnp.float32),
            pltpu.SemaphoreType.DMA((NIG,)),
            pltpu.SemaphoreType.DMA((NRB,)),
            pltpu.SemaphoreType.DMA((NRB,)),
        ],
    )


def _bn_relu(z, g, b):
    mu = jnp.mean(z, axis=0, keepdims=True)
    zc = z - mu
    var = jnp.mean(zc * zc, axis=0, keepdims=True)
    return jnp.maximum(zc / jnp.sqrt(var + 1e-5) * g + b, 0.0)


def _tc_layer_body(h_b, p_b, w1_b, b1_b, g1_b, be1_b, w2_b, b2_b, g2_b,
                   be2_b, bat_b, hout_b, ro_b):
    h = h_b[...]
    z = p_b[0] + p_b[1] - h
    z = jnp.dot(z, w1_b[...], preferred_element_type=jnp.float32) + b1_b[...]
    z = _bn_relu(z, g1_b[...], be1_b[...])
    z = jnp.dot(z, w2_b[...], preferred_element_type=jnp.float32) + b2_b[...]
    hn = _bn_relu(z, g2_b[...], be2_b[...])
    hout_b[...] = hn
    g = ro_b.shape[0]
    seg = lax.broadcasted_iota(jnp.int32, (g, h.shape[0]), 0)
    onehot = (seg == bat_b[...]).astype(jnp.float32)
    ro_b[...] = jnp.dot(onehot, hn, preferred_element_type=jnp.float32)


@functools.lru_cache(maxsize=None)
def _make_tc_layer(n, d, hdim, g):
    return pl.pallas_call(
        _tc_layer_body,
        out_shape=(jax.ShapeDtypeStruct((n, hdim), jnp.float32),
                   jax.ShapeDtypeStruct((g, hdim), jnp.float32)),
    )


def _tc_final_body(h_b, p_b, w1_b, b1_b, g1_b, be1_b, w2_b, b2_b, g2_b,
                   be2_b, bat_b, r0_b, r1_b, wc1_b, bc1_b, wc2_b, bc2_b,
                   out_b):
    h = h_b[...]
    z = p_b[0] + p_b[1] - h
    z = jnp.dot(z, w1_b[...], preferred_element_type=jnp.float32) + b1_b[...]
    z = _bn_relu(z, g1_b[...], be1_b[...])
    z = jnp.dot(z, w2_b[...], preferred_element_type=jnp.float32) + b2_b[...]
    hn = _bn_relu(z, g2_b[...], be2_b[...])
    g = out_b.shape[0]
    seg = lax.broadcasted_iota(jnp.int32, (g, h.shape[0]), 0)
    onehot = (seg == bat_b[...]).astype(jnp.float32)
    r2 = jnp.dot(onehot, hn, preferred_element_type=jnp.float32)
    hdim = r0_b.shape[1]
    hc = (jnp.dot(r0_b[...], wc1_b[0:hdim], preferred_element_type=jnp.float32)
          + jnp.dot(r1_b[...], wc1_b[hdim:2 * hdim],
                    preferred_element_type=jnp.float32)
          + jnp.dot(r2, wc1_b[2 * hdim:3 * hdim],
                    preferred_element_type=jnp.float32))
    hc = jnp.maximum(hc + bc1_b[...], 0.0)
    out_b[...] = (jnp.dot(hc, wc2_b[...], preferred_element_type=jnp.float32)
                  + bc2_b[...])


@functools.lru_cache(maxsize=None)
def _make_tc_final(g, c):
    return pl.pallas_call(
        _tc_final_body,
        out_shape=jax.ShapeDtypeStruct((g, c), jnp.float32),
    )


def kernel(x, params, edge_index, batch):
    n, d = x.shape
    e = edge_index.shape[1]
    g = 64
    nlayers = sum(1 for k in params if k.startswith('layer'))
    c_out = params['Wc2'].shape[1]
    hdim = params['layer0']['W2'].shape[1]

    sc_agg = _make_sc_agg(n, d, e)
    tc_layer = _make_tc_layer(n, d, hdim, g)
    tc_final = _make_tc_final(g, c_out)

    bat2d = batch.reshape(1, n)
    nrounds = e // (NC * NS) // CHUNK
    idx4 = jnp.transpose(
        edge_index.reshape(2, NC * NS, nrounds, CHUNK), (1, 2, 0, 3))
    h = x
    ros = []
    for i in range(nlayers - 1):
        p = params['layer%d' % i]
        part = sc_agg(h, idx4)
        h, ro = tc_layer(h, part,
                         p['W1'], p['b1'].reshape(1, -1),
                         p['g1'].reshape(1, -1), p['be1'].reshape(1, -1),
                         p['W2'], p['b2'].reshape(1, -1),
                         p['g_out'].reshape(1, -1), p['be_out'].reshape(1, -1),
                         bat2d)
        ros.append(ro)
    p = params['layer%d' % (nlayers - 1)]
    part = sc_agg(h, idx4)
    return tc_final(h, part,
                    p['W1'], p['b1'].reshape(1, -1),
                    p['g1'].reshape(1, -1), p['be1'].reshape(1, -1),
                    p['W2'], p['b2'].reshape(1, -1),
                    p['g_out'].reshape(1, -1), p['be_out'].reshape(1, -1),
                    bat2d, ros[0], ros[1],
                    params['Wc1'], params['bc1'].reshape(1, -1),
                    params['Wc2'], params['bc2'].reshape(1, -1))
